# baseline (speedup 1.0000x reference)
import functools

import jax
import jax.numpy as jnp
from jax import lax
from jax.experimental import pallas as pl
from jax.experimental.pallas import tpu as pltpu
from jax.experimental.pallas import tpu_sc as plsc

N_CLS = 1000
N_CTX = 16
D = 512
VOCAB = 49408
M = N_CLS * N_CTX
M_PAD = 16384
V_PAD = 49664
TM = 2048
TV = 512
NM = M_PAD // TM
NV = V_PAD // TV

SC_NC = 2
SC_NS = 16
SC_NW = SC_NC * SC_NS
B_PER_W = M_PAD // SC_NW
CHUNK = 128
NCHUNK = B_PER_W // CHUNK


def _argmin_body(a_ref, b_ref, o_ref, best_ref, bidx_ref):
    v = pl.program_id(1)
    a = a_ref[...]
    b = b_ref[...]
    s = lax.dot_general(a, b, (((1,), (1,)), ((), ())),
                        preferred_element_type=jnp.float32)
    ones = jnp.ones((1, D), dtype=b.dtype)
    b2 = lax.dot_general(ones, b * b, (((1,), (1,)), ((), ())),
                         preferred_element_type=jnp.float32)
    total = 0.5 * b2 - s
    local_min = jnp.min(total, axis=1, keepdims=True)
    lane = lax.broadcasted_iota(jnp.int32, total.shape, 1)
    local_arg = jnp.min(jnp.where(total == local_min, lane, TV),
                        axis=1, keepdims=True) + v * TV

    @pl.when(v == 0)
    def _():
        best_ref[...] = local_min
        bidx_ref[...] = local_arg

    @pl.when(v > 0)
    def _():
        better = local_min < best_ref[...]
        bidx_ref[...] = jnp.where(better, local_arg, bidx_ref[...])
        best_ref[...] = jnp.where(better, local_min, best_ref[...])

    @pl.when(v == NV - 1)
    def _():
        o_ref[...] = bidx_ref[...]


def _nearest_indices(a_pad, b_pad):
    return pl.pallas_call(
        _argmin_body,
        grid=(NM, NV),
        in_specs=[
            pl.BlockSpec((TM, D), lambda m, v: (m, 0)),
            pl.BlockSpec((TV, D), lambda m, v: (v, 0)),
        ],
        out_specs=pl.BlockSpec((TM, 1), lambda m, v: (m, 0)),
        out_shape=jax.ShapeDtypeStruct((M_PAD, 1), jnp.int32),
        scratch_shapes=[
            pltpu.VMEM((TM, 1), jnp.float32),
            pltpu.VMEM((TM, 1), jnp.int32),
        ],
        compiler_params=pltpu.CompilerParams(
            dimension_semantics=("parallel", "arbitrary"),
        ),
    )(a_pad, b_pad)


def _sc_gather_body(table_hbm, idx_hbm, out_hbm, idx_v, rows_v, sem):
    wid = lax.axis_index("s") * SC_NC + lax.axis_index("c")
    for c in range(NCHUNK):
        base = wid * B_PER_W + c * CHUNK
        pltpu.sync_copy(idx_hbm.at[pl.ds(base, CHUNK)], idx_v)
        pltpu.async_copy(table_hbm.at[idx_v], rows_v, sem).wait()
        pltpu.sync_copy(rows_v, out_hbm.at[pl.ds(base, CHUNK)])


@functools.cache
def _sc_gather():
    return pl.kernel(
        _sc_gather_body,
        out_type=jax.ShapeDtypeStruct((M_PAD, D), jnp.float32),
        mesh=plsc.VectorSubcoreMesh(core_axis_name="c", subcore_axis_name="s"),
        scratch_types=[
            pltpu.VMEM((CHUNK,), jnp.int32),
            pltpu.VMEM((CHUNK, D), jnp.float32),
            pltpu.SemaphoreType.DMA,
        ],
    )


def kernel(ctx, token_embedding, token_prefix, token_suffix):
    a = ctx.reshape(M, D)
    a_pad = jnp.pad(a, ((0, M_PAD - M), (0, 0))).astype(jnp.bfloat16)
    b_pad = jnp.pad(token_embedding, ((0, V_PAD - VOCAB), (0, 0)),
                    constant_values=100.0).astype(jnp.bfloat16)
    idx = _nearest_indices(a_pad, b_pad).reshape(M_PAD)
    projected_flat = _sc_gather()(token_embedding, idx)
    projected = projected_flat[:M].reshape(N_CLS, N_CTX, D)
    return jnp.concatenate([token_prefix, projected, token_suffix], axis=1)


# speedup vs baseline: 1.6546x; 1.6546x over previous
import functools

import jax
import jax.numpy as jnp
from jax import lax
from jax.experimental import pallas as pl
from jax.experimental.pallas import tpu as pltpu
from jax.experimental.pallas import tpu_sc as plsc

N_CLS = 1000
N_CTX = 16
D = 512
VOCAB = 49408
M = N_CLS * N_CTX
M_PAD = 16384
V_PAD = 49664
TM = 2048
TV = 512
NM = M_PAD // TM
NV = V_PAD // TV

SC_NC = 2
SC_NS = 16
SC_NW = SC_NC * SC_NS
B_PER_W = M_PAD // SC_NW
CHUNK = 128
NCHUNK = B_PER_W // CHUNK


CR = 128
CC = 128
LANE_BITS = 511


def _c2_body(b_ref, o_ref):
    b = b_ref[...]
    ones = jnp.ones((1, D), dtype=b.dtype)
    b2 = lax.dot_general(ones, b * b, (((1,), (1,)), ((), ())),
                         preferred_element_type=jnp.float32)
    o_ref[0] = 0.5 * b2 + 1.0


def _c2_rows(b_pad):
    return pl.pallas_call(
        _c2_body,
        grid=(NV,),
        in_specs=[pl.BlockSpec((TV, D), lambda v: (v, 0))],
        out_specs=pl.BlockSpec((1, 1, TV), lambda v: (v, 0, 0)),
        out_shape=jax.ShapeDtypeStruct((NV, 1, TV), jnp.float32),
    )(b_pad)


def _phase(v, a_ref, b_ref, c2_ref, o_ref, sw_ref, sr_ref,
           best_ref, bidx_ref):
    a = a_ref[...]
    b = b_ref[...]
    u = v - 1
    u0 = u == 0
    lanes = [
        lax.broadcasted_iota(jnp.int32, (CR, CC), 1) + c * CC
        for c in range(TV // CC)
    ]
    locals_ = []
    nsplit = 2
    nw = TV // nsplit
    rper = (TM // CR) // nsplit
    for half in range(nsplit):
        cs = slice(half * nw, (half + 1) * nw)
        sw_ref[:, cs] = lax.dot_general(
            a, b[cs, :], (((1,), (1,)), ((), ())),
            preferred_element_type=jnp.float32)
        for r in range(half * rper, (half + 1) * rper):
            rs = slice(r * CR, (r + 1) * CR)
            ks = []
            for c in range(TV // CC):
                y = sr_ref[rs, c * CC:(c + 1) * CC] \
                    + c2_ref[0, :, c * CC:(c + 1) * CC]
                bits = jax.lax.bitcast_convert_type(y, jnp.int32)
                ks.append(jax.lax.bitcast_convert_type(
                    (bits & ~LANE_BITS) | lanes[c], jnp.float32))
            while len(ks) > 1:
                ks = [jnp.minimum(ks[i], ks[i + 1])
                      for i in range(0, len(ks), 2)]
            locals_.append(jnp.min(ks[0], axis=1))

    local = jnp.stack(locals_)
    lbits = jax.lax.bitcast_convert_type(local, jnp.int32)
    gidx = (lbits & LANE_BITS) + u * TV

    @pl.when(v > 0)
    def _():
        better = jnp.logical_or(local < best_ref[...], u0)
        bidx_ref[...] = jnp.where(better, gidx, bidx_ref[...])
        best_ref[...] = jnp.where(better, local, best_ref[...])

    @pl.when(v == NV)
    def _():
        o_ref[0] = bidx_ref[...]


def _argmin_body(a_ref, b_ref, c2_ref, o_ref, s0_ref, s1_ref,
                 best_ref, bidx_ref):
    v = pl.program_id(1)
    p = lax.rem(v, 2)

    @pl.when(p == 0)
    def _():
        _phase(v, a_ref, b_ref, c2_ref, o_ref, s0_ref, s1_ref,
               best_ref, bidx_ref)

    @pl.when(p == 1)
    def _():
        _phase(v, a_ref, b_ref, c2_ref, o_ref, s1_ref, s0_ref,
               best_ref, bidx_ref)


def _nearest_indices(a_pad, b_pad):
    c2_all = _c2_rows(b_pad)
    out = pl.pallas_call(
        _argmin_body,
        grid=(NM, NV + 1),
        in_specs=[
            pl.BlockSpec((TM, D), lambda m, v: (m, 0)),
            pl.BlockSpec((TV, D), lambda m, v: (jnp.minimum(v, NV - 1), 0)),
            pl.BlockSpec((1, 1, TV),
                         lambda m, v: (jnp.maximum(v - 1, 0), 0, 0)),
        ],
        out_specs=pl.BlockSpec((1, TM // CR, CR), lambda m, v: (m, 0, 0)),
        out_shape=jax.ShapeDtypeStruct((NM, TM // CR, CR), jnp.int32),
        scratch_shapes=[
            pltpu.VMEM((TM, TV), jnp.float32),
            pltpu.VMEM((TM, TV), jnp.float32),
            pltpu.VMEM((TM // CR, CR), jnp.float32),
            pltpu.VMEM((TM // CR, CR), jnp.int32),
        ],
        compiler_params=pltpu.CompilerParams(
            dimension_semantics=("parallel", "arbitrary"),
        ),
    )(a_pad, b_pad, c2_all)
    return out


def _sc_gather_body(table_hbm, idx_hbm, out_hbm, idx_v, rows_v, sem):
    wid = lax.axis_index("s") * SC_NC + lax.axis_index("c")
    for c in range(NCHUNK):
        base = wid * B_PER_W + c * CHUNK
        pltpu.sync_copy(idx_hbm.at[pl.ds(base, CHUNK)], idx_v)
        pltpu.async_copy(table_hbm.at[idx_v], rows_v, sem).wait()
        pltpu.sync_copy(rows_v, out_hbm.at[pl.ds(base, CHUNK)])


@functools.cache
def _sc_gather():
    return pl.kernel(
        _sc_gather_body,
        out_type=jax.ShapeDtypeStruct((M_PAD, D), jnp.float32),
        mesh=plsc.VectorSubcoreMesh(core_axis_name="c", subcore_axis_name="s"),
        scratch_types=[
            pltpu.VMEM((CHUNK,), jnp.int32),
            pltpu.VMEM((CHUNK, D), jnp.float32),
            pltpu.SemaphoreType.DMA,
        ],
    )


def kernel(ctx, token_embedding, token_prefix, token_suffix):
    a = ctx.reshape(M, D)
    a_pad = jnp.pad(-a, ((0, M_PAD - M), (0, 0))).astype(jnp.bfloat16)
    b_pad = jnp.pad(token_embedding, ((0, V_PAD - VOCAB), (0, 0)),
                    constant_values=100.0).astype(jnp.bfloat16)
    idx = _nearest_indices(a_pad, b_pad).reshape(M_PAD)
    projected_flat = _sc_gather()(token_embedding, idx)
    projected = projected_flat[:M].reshape(N_CLS, N_CTX, D)
    return jnp.concatenate([token_prefix, projected, token_suffix], axis=1)


# speedup vs baseline: 1.7143x; 1.0361x over previous
import functools

import jax
import jax.numpy as jnp
from jax import lax
from jax.experimental import pallas as pl
from jax.experimental.pallas import tpu as pltpu
from jax.experimental.pallas import tpu_sc as plsc

N_CLS = 1000
N_CTX = 16
D = 512
VOCAB = 49408
M = N_CLS * N_CTX
M_PAD = 16384
V_PAD = 50176
TM = 2048
TV = 1024
NM = M_PAD // TM
NV = V_PAD // TV

SC_NC = 2
SC_NS = 16
SC_NW = SC_NC * SC_NS
B_PER_W = M_PAD // SC_NW
CHUNK = 128
NCHUNK = B_PER_W // CHUNK


CR = 128
CC = 128
LANE_BITS = 1023


def _c2_body(b_ref, o_ref):
    b = b_ref[...]
    ones = jnp.ones((1, D), dtype=b.dtype)
    b2 = lax.dot_general(ones, b * b, (((1,), (1,)), ((), ())),
                         preferred_element_type=jnp.float32)
    o_ref[0] = 0.5 * b2 + 1.0


def _c2_rows(b_pad):
    return pl.pallas_call(
        _c2_body,
        grid=(NV,),
        in_specs=[pl.BlockSpec((TV, D), lambda v: (v, 0))],
        out_specs=pl.BlockSpec((1, 1, TV), lambda v: (v, 0, 0)),
        out_shape=jax.ShapeDtypeStruct((NV, 1, TV), jnp.float32),
    )(b_pad)


def _phase(v, a_ref, b_ref, c2_ref, o_ref, sw_ref, sr_ref,
           best_ref, bidx_ref):
    a = a_ref[...]
    b = b_ref[...]
    u = v - 1
    u0 = u == 0
    lanes = [
        lax.broadcasted_iota(jnp.int32, (CR, CC), 1) + c * CC
        for c in range(TV // CC)
    ]
    locals_ = []
    nsplit = 4
    nw = TV // nsplit
    rper = (TM // CR) // nsplit
    for half in range(nsplit):
        cs = slice(half * nw, (half + 1) * nw)
        sw_ref[:, cs] = lax.dot_general(
            a, b[cs, :], (((1,), (1,)), ((), ())),
            preferred_element_type=jnp.float32)
        for r in range(half * rper, (half + 1) * rper):
            rs = slice(r * CR, (r + 1) * CR)
            ks = []
            for c in range(TV // CC):
                y = sr_ref[rs, c * CC:(c + 1) * CC] \
                    + c2_ref[0, :, c * CC:(c + 1) * CC]
                bits = jax.lax.bitcast_convert_type(y, jnp.int32)
                ks.append(jax.lax.bitcast_convert_type(
                    (bits & ~LANE_BITS) | lanes[c], jnp.float32))
            while len(ks) > 1:
                ks = [jnp.minimum(ks[i], ks[i + 1])
                      for i in range(0, len(ks), 2)]
            locals_.append(jnp.min(ks[0], axis=1))

    local = jnp.stack(locals_)
    lbits = jax.lax.bitcast_convert_type(local, jnp.int32)
    gidx = (lbits & LANE_BITS) + u * TV

    @pl.when(v > 0)
    def _():
        better = jnp.logical_or(local < best_ref[...], u0)
        bidx_ref[...] = jnp.where(better, gidx, bidx_ref[...])
        best_ref[...] = jnp.where(better, local, best_ref[...])

    @pl.when(v == NV)
    def _():
        o_ref[0] = bidx_ref[...]


def _argmin_body(a_ref, b_ref, c2_ref, o_ref, s0_ref, s1_ref,
                 best_ref, bidx_ref):
    v = pl.program_id(1)
    p = lax.rem(v, 2)

    @pl.when(p == 0)
    def _():
        _phase(v, a_ref, b_ref, c2_ref, o_ref, s0_ref, s1_ref,
               best_ref, bidx_ref)

    @pl.when(p == 1)
    def _():
        _phase(v, a_ref, b_ref, c2_ref, o_ref, s1_ref, s0_ref,
               best_ref, bidx_ref)


def _nearest_indices(a_pad, b_pad):
    c2_all = _c2_rows(b_pad)
    out = pl.pallas_call(
        _argmin_body,
        grid=(NM, NV + 1),
        in_specs=[
            pl.BlockSpec((TM, D), lambda m, v: (m, 0)),
            pl.BlockSpec((TV, D), lambda m, v: (jnp.minimum(v, NV - 1), 0)),
            pl.BlockSpec((1, 1, TV),
                         lambda m, v: (jnp.maximum(v - 1, 0), 0, 0)),
        ],
        out_specs=pl.BlockSpec((1, TM // CR, CR), lambda m, v: (m, 0, 0)),
        out_shape=jax.ShapeDtypeStruct((NM, TM // CR, CR), jnp.int32),
        scratch_shapes=[
            pltpu.VMEM((TM, TV), jnp.float32),
            pltpu.VMEM((TM, TV), jnp.float32),
            pltpu.VMEM((TM // CR, CR), jnp.float32),
            pltpu.VMEM((TM // CR, CR), jnp.int32),
        ],
        compiler_params=pltpu.CompilerParams(
            dimension_semantics=("parallel", "arbitrary"),
        ),
    )(a_pad, b_pad, c2_all)
    return out


def _sc_gather_body(table_hbm, idx_hbm, out_hbm, idx_v, rows_v, sem):
    wid = lax.axis_index("s") * SC_NC + lax.axis_index("c")
    for c in range(NCHUNK):
        base = wid * B_PER_W + c * CHUNK
        pltpu.sync_copy(idx_hbm.at[pl.ds(base, CHUNK)], idx_v)
        pltpu.async_copy(table_hbm.at[idx_v], rows_v, sem).wait()
        pltpu.sync_copy(rows_v, out_hbm.at[pl.ds(base, CHUNK)])


@functools.cache
def _sc_gather():
    return pl.kernel(
        _sc_gather_body,
        out_type=jax.ShapeDtypeStruct((M_PAD, D), jnp.float32),
        mesh=plsc.VectorSubcoreMesh(core_axis_name="c", subcore_axis_name="s"),
        scratch_types=[
            pltpu.VMEM((CHUNK,), jnp.int32),
            pltpu.VMEM((CHUNK, D), jnp.float32),
            pltpu.SemaphoreType.DMA,
        ],
    )


def kernel(ctx, token_embedding, token_prefix, token_suffix):
    a = ctx.reshape(M, D)
    a_pad = jnp.pad(-a, ((0, M_PAD - M), (0, 0))).astype(jnp.bfloat16)
    b_pad = jnp.pad(token_embedding, ((0, V_PAD - VOCAB), (0, 0)),
                    constant_values=100.0).astype(jnp.bfloat16)
    idx = _nearest_indices(a_pad, b_pad).reshape(M_PAD)
    projected_flat = _sc_gather()(token_embedding, idx)
    projected = projected_flat[:M].reshape(N_CLS, N_CTX, D)
    return jnp.concatenate([token_prefix, projected, token_suffix], axis=1)
